# Initial kernel scaffold; baseline (speedup 1.0000x reference)
import functools

import jax
import jax.numpy as jnp
from jax import lax
from jax.experimental import pallas as pl
from jax.experimental.pallas import tpu as pltpu
from jax.experimental.pallas import tpu_sc as plsc

B = 16384
D = 64
K = 20

NC = 2
NS = 16
NW = NC * NS
BPW = B // NW
CB = 64
NCHUNK = BPW // CB
GSUB = 128


def _sc_body(in_hbm, out_hbm, cidx_hbm, pidx_hbm, nidx_hbm,
             pos_hbm, neg_hbm,
             cidx_v, pidx_v, nidx_v, crows, prows, nrows,
             pos_out, neg_out, sem):
    wid = lax.axis_index("s") * NC + lax.axis_index("c")
    base = wid * BPW

    pltpu.sync_copy(cidx_hbm.at[pl.ds(base, BPW)], cidx_v)
    pltpu.sync_copy(pidx_hbm.at[pl.ds(base, BPW)], pidx_v)
    pltpu.sync_copy(nidx_hbm.at[pl.ds(base * K, BPW * K)], nidx_v)

    for c in range(NCHUNK):
        copies = [
            pltpu.make_async_copy(
                in_hbm.at[cidx_v.at[pl.ds(c * CB, CB)]], crows, sem),
            pltpu.make_async_copy(
                out_hbm.at[pidx_v.at[pl.ds(c * CB, CB)]], prows, sem),
        ]
        for i in range(CB * K // GSUB):
            copies.append(pltpu.make_async_copy(
                out_hbm.at[nidx_v.at[pl.ds(c * CB * K + i * GSUB, GSUB)]],
                nrows.at[pl.ds(i * GSUB, GSUB), :], sem))
        for cp in copies:
            cp.start()
        for cp in copies:
            cp.wait()

        def bbody(j, carry):
            c0 = crows[j, pl.ds(0, 16)]
            c1 = crows[j, pl.ds(16, 16)]
            c2 = crows[j, pl.ds(32, 16)]
            c3 = crows[j, pl.ds(48, 16)]
            p0 = prows[j, pl.ds(0, 16)]
            p1 = prows[j, pl.ds(16, 16)]
            p2 = prows[j, pl.ds(32, 16)]
            p3 = prows[j, pl.ds(48, 16)]
            acc = c0 * p0 + c1 * p1 + c2 * p2 + c3 * p3
            pos_out[c * CB + j] = jnp.sum(acc)
            for k in range(K):
                r = j * K + k
                n0 = nrows[r, pl.ds(0, 16)]
                n1 = nrows[r, pl.ds(16, 16)]
                n2 = nrows[r, pl.ds(32, 16)]
                n3 = nrows[r, pl.ds(48, 16)]
                accn = c0 * n0 + c1 * n1 + c2 * n2 + c3 * n3
                neg_out[(c * CB + j) * K + k] = jnp.sum(accn)
            return carry

        lax.fori_loop(0, CB, bbody, 0)

    pltpu.sync_copy(pos_out, pos_hbm.at[pl.ds(base, BPW)])
    pltpu.sync_copy(neg_out, neg_hbm.at[pl.ds(base * K, BPW * K)])


_mesh = plsc.VectorSubcoreMesh(
    core_axis_name="c", subcore_axis_name="s", num_cores=NC, num_subcores=NS)

_sc_call = pl.kernel(
    _sc_body,
    out_type=(
        jax.ShapeDtypeStruct((B,), jnp.float32),
        jax.ShapeDtypeStruct((B * K,), jnp.float32),
    ),
    mesh=_mesh,
    scratch_types=[
        pltpu.VMEM((BPW,), jnp.int32),
        pltpu.VMEM((BPW,), jnp.int32),
        pltpu.VMEM((BPW * K,), jnp.int32),
        pltpu.VMEM((CB, D), jnp.float32),
        pltpu.VMEM((CB, D), jnp.float32),
        pltpu.VMEM((CB * K, D), jnp.float32),
        pltpu.VMEM((BPW,), jnp.float32),
        pltpu.VMEM((BPW * K,), jnp.float32),
        pltpu.SemaphoreType.DMA,
    ],
)


@jax.jit
def kernel(center_words, pos_context_words, neg_context_words,
           in_table, out_table):
    cidx = center_words.astype(jnp.int32)
    pidx = pos_context_words.astype(jnp.int32)
    nidx = neg_context_words.astype(jnp.int32).reshape(-1)
    pos_score, neg_score = _sc_call(in_table, out_table, cidx, pidx, nidx)
    return pos_score, neg_score.reshape(B, K)


# speedup vs baseline: 2.4421x; 2.4421x over previous
import functools

import jax
import jax.numpy as jnp
from jax import lax
from jax.experimental import pallas as pl
from jax.experimental.pallas import tpu as pltpu
from jax.experimental.pallas import tpu_sc as plsc

B = 16384
D = 64
K = 20

NC = 2
NS = 16
NW = NC * NS
BPW = B // NW
CB = 64
NCHUNK = BPW // CB
GSUB = 128


def _sc_body(in_hbm, out_hbm, cidx_hbm, pidx_hbm, nidx_hbm,
             pos_hbm, neg_hbm,
             cidx_v, pidx_v, nidx_v, crows, prows, nrows,
             pos_out, neg_out, sem):
    wid = lax.axis_index("s") * NC + lax.axis_index("c")
    base = wid * BPW

    pltpu.sync_copy(cidx_hbm.at[pl.ds(base, BPW)], cidx_v)
    pltpu.sync_copy(pidx_hbm.at[pl.ds(base, BPW)], pidx_v)
    pltpu.sync_copy(nidx_hbm.at[pl.ds(base * K, BPW * K)], nidx_v)

    for c in range(NCHUNK):
        copies = [
            pltpu.make_async_copy(
                in_hbm.at[cidx_v.at[pl.ds(c * CB, CB)]], crows, sem),
            pltpu.make_async_copy(
                out_hbm.at[pidx_v.at[pl.ds(c * CB, CB)]], prows, sem),
        ]
        for i in range(CB * K // GSUB):
            copies.append(pltpu.make_async_copy(
                out_hbm.at[nidx_v.at[pl.ds(c * CB * K + i * GSUB, GSUB)]],
                nrows.at[pl.ds(i * GSUB, GSUB), :], sem))
        for cp in copies:
            cp.start()
        for cp in copies:
            cp.wait()

        for g in range(CB // 16):
            b_ids = lax.iota(jnp.int32, 16) + (g * 16)
            rk = [b_ids * K + k for k in range(K)]
            zero = jnp.zeros((16,), jnp.float32)

            def dbody(d, accs):
                dsplat = jnp.full((16,), d, jnp.int32)
                cd = plsc.load_gather(crows, [b_ids, dsplat])
                pd = plsc.load_gather(prows, [b_ids, dsplat])
                acc_p = accs[0] + cd * pd
                acc_n = [
                    accs[1 + k] + cd * plsc.load_gather(nrows, [rk[k], dsplat])
                    for k in range(K)
                ]
                return (acc_p, *acc_n)

            accs = lax.fori_loop(0, D, dbody, (zero,) * (K + 1))
            off = c * CB + g * 16
            pos_out[pl.ds(off, 16)] = accs[0]
            for k in range(K):
                neg_out[k, pl.ds(off, 16)] = accs[1 + k]

    pltpu.sync_copy(pos_out, pos_hbm.at[pl.ds(base, BPW)])
    for k in range(K):
        pltpu.sync_copy(neg_out.at[k], neg_hbm.at[pl.ds(k * B + base, BPW)])


_mesh = plsc.VectorSubcoreMesh(
    core_axis_name="c", subcore_axis_name="s", num_cores=NC, num_subcores=NS)

_sc_call = pl.kernel(
    _sc_body,
    out_type=(
        jax.ShapeDtypeStruct((B,), jnp.float32),
        jax.ShapeDtypeStruct((K * B,), jnp.float32),
    ),
    mesh=_mesh,
    compiler_params=pltpu.CompilerParams(
        needs_layout_passes=False, use_tc_tiling_on_sc=False),
    scratch_types=[
        pltpu.VMEM((BPW,), jnp.int32),
        pltpu.VMEM((BPW,), jnp.int32),
        pltpu.VMEM((BPW * K,), jnp.int32),
        pltpu.VMEM((CB, D), jnp.float32),
        pltpu.VMEM((CB, D), jnp.float32),
        pltpu.VMEM((CB * K, D), jnp.float32),
        pltpu.VMEM((BPW,), jnp.float32),
        pltpu.VMEM((K, BPW), jnp.float32),
        pltpu.SemaphoreType.DMA,
    ],
)


@jax.jit
def kernel(center_words, pos_context_words, neg_context_words,
           in_table, out_table):
    cidx = center_words.astype(jnp.int32)
    pidx = pos_context_words.astype(jnp.int32)
    nidx = neg_context_words.astype(jnp.int32).reshape(-1)
    pos_score, neg_score = _sc_call(in_table, out_table, cidx, pidx, nidx)
    return pos_score, neg_score.reshape(K, B).T
